# baseline (speedup 1.0000x reference)
import functools

import jax
import jax.numpy as jnp
from jax import lax
from jax.experimental import pallas as pl
from jax.experimental.pallas import tpu as pltpu
from jax.experimental.pallas import tpu_sc as plsc

N = 50000
D = 128
E = 310000
SW = 32
NSLAB = 4

GT = 128
NG = 156
IBG = 12
NCH = NG // IBG
NBUF = 4
PT = NG * GT
EP = 16 * PT
TPT = 3200
ACC = 16 * TPT

DPT = 19456
EPD = 16 * DPT
NP = 65536
NPT = NP // 16

BR = 512
GRID = (N + BR - 1) // BR

_mesh = plsc.VectorSubcoreMesh(core_axis_name="c", subcore_axis_name="s")
_sc_params = pltpu.CompilerParams(needs_layout_passes=False,
                                  use_tc_tiling_on_sc=False)
_f32 = jnp.float32
_i32 = jnp.int32


@functools.partial(
    pl.kernel,
    out_type=(jax.ShapeDtypeStruct((4 * NP,), _f32),
              jax.ShapeDtypeStruct((64 * NP,), _f32)),
    mesh=_mesh,
    compiler_params=_sc_params,
    scratch_types=[
        pltpu.VMEM((NP,), _f32),
        pltpu.VMEM((DPT,), _i32),
        pltpu.VMEM((NPT,), _f32),
        pltpu.VMEM((NPT,), _f32),
    ],
)
def _degrees(degidx_hbm, degout_hbm, part_hbm, cnt, idxb, acc, tmp):
    c = lax.axis_index("c")
    s = lax.axis_index("s")
    zeros16 = jnp.zeros((16,), _f32)
    ones16 = jnp.ones((16,), _f32)
    for q in range(2):
        p = 2 * c + q

        def zero_body(i, _):
            cnt[pl.ds(i * 16, 16)] = zeros16
            return 0

        lax.fori_loop(0, NP // 16, zero_body, 0)
        pltpu.sync_copy(degidx_hbm.at[pl.ds((p * 16 + s) * DPT, DPT)], idxb)

        def scat_body(j, _):
            idx16 = idxb[pl.ds(j * 16, 16)]
            plsc.addupdate_scatter(cnt, [idx16], ones16)
            return 0

        lax.fori_loop(0, DPT // 16, scat_body, 0)
        pltpu.sync_copy(cnt, part_hbm.at[pl.ds((p * 16 + s) * NP, NP)])
        plsc.subcore_barrier()
        pltpu.sync_copy(part_hbm.at[pl.ds((p * 16) * NP + s * NPT, NPT)], acc)
        for u in range(1, 16):
            pltpu.sync_copy(
                part_hbm.at[pl.ds((p * 16 + u) * NP + s * NPT, NPT)], tmp)

            def red_body(i, _):
                sl = pl.ds(i * 16, 16)
                acc[sl] = acc[sl] + tmp[sl]
                return 0

            lax.fori_loop(0, NPT // 16, red_body, 0)
        pltpu.sync_copy(acc, degout_hbm.at[pl.ds(p * NP + s * NPT, NPT)])


@functools.partial(
    pl.kernel,
    out_type=(jax.ShapeDtypeStruct((ACC, D), _f32),
              jax.ShapeDtypeStruct((ACC, D), _f32)),
    mesh=_mesh,
    compiler_params=_sc_params,
    scratch_types=[
        pltpu.VMEM((IBG, GT), _i32),
        pltpu.VMEM((IBG, GT), _i32),
        pltpu.VMEM((NBUF, GT, SW), _f32),
        pltpu.VMEM((GT, SW), _f32),
        pltpu.VMEM_SHARED((ACC, SW), _f32),
    ] + [pltpu.SemaphoreType.DMA] * NBUF,
)
def _aggregate(feat0, feat1, gidx, dstidx, zrows, outA, outB,
               gbuf, dbuf, rows, zb, sacc, *sems):
    c = lax.axis_index("c")
    s = lax.axis_index("s")
    pltpu.sync_copy(zrows, zb)
    for q in range(4):
        r, si = q // 2, q % 2
        slab = 2 * c + si
        feat = feat0 if r == 0 else feat1
        out = outA if r == 0 else outB

        def zero_body(j, _):
            pltpu.sync_copy(zb, sacc.at[pl.ds(s * TPT + j * GT, GT)])
            return 0

        lax.fori_loop(0, TPT // GT, zero_body, 0)
        plsc.subcore_barrier()

        def chunk_body(ch, _):
            pltpu.sync_copy(gidx.at[r, slab, s, pl.ds(ch * IBG, IBG)], gbuf)
            pltpu.sync_copy(dstidx.at[r, s, pl.ds(ch * IBG, IBG)], dbuf)
            descs = [None] * IBG
            for g in range(NBUF):
                descs[g] = pltpu.async_copy(
                    feat.at[gbuf.at[g]], rows.at[g], sems[g])
            for g in range(IBG):
                b = g % NBUF
                descs[g].wait()
                pltpu.sync_copy(rows.at[b], sacc.at[dbuf.at[g]], add=True)
                if g + NBUF < IBG:
                    descs[g + NBUF] = pltpu.async_copy(
                        feat.at[gbuf.at[g + NBUF]], rows.at[b], sems[b])
            return 0

        lax.fori_loop(0, NCH, chunk_body, 0)
        plsc.subcore_barrier()

        def out_body(j, _):
            pltpu.sync_copy(sacc.at[pl.ds(s * TPT + j * GT, GT)],
                            out.at[pl.ds(s * TPT + j * GT, GT),
                                   pl.ds(slab * SW, SW)])
            return 0

        lax.fori_loop(0, TPT // GT, out_body, 0)
        plsc.subcore_barrier()


def _featscale_body(x_ref, ns0_ref, ns1_ref, f0_ref, f1_ref):
    xb = x_ref[...]
    f0_ref[...] = xb * ns0_ref[:, :1]
    f1_ref[...] = xb * ns1_ref[:, :1]


_featscale = pl.pallas_call(
    _featscale_body,
    grid=(GRID,),
    in_specs=[
        pl.BlockSpec((BR, D), lambda i: (i, 0)),
        pl.BlockSpec((BR, SW), lambda i: (i, 0)),
        pl.BlockSpec((BR, SW), lambda i: (i, 0)),
    ],
    out_specs=[
        pl.BlockSpec((BR, D), lambda i: (i, 0)),
        pl.BlockSpec((BR, D), lambda i: (i, 0)),
    ],
    out_shape=[
        jax.ShapeDtypeStruct((N, D), _f32),
        jax.ShapeDtypeStruct((N, D), _f32),
    ],
)


def _mm_block(aggA_ref, aggB_ref, nd0_ref, nd1_ref, w0_ref, w1_ref, bs_ref):
    acc = jnp.broadcast_to(bs_ref[...], (BR, D))
    acc = acc + jnp.dot(aggA_ref[...] * nd0_ref[:, :1], w0_ref[...],
                        precision=lax.Precision.HIGHEST,
                        preferred_element_type=_f32)
    acc = acc + jnp.dot(aggB_ref[...] * nd1_ref[:, :1], w1_ref[...],
                        precision=lax.Precision.HIGHEST,
                        preferred_element_type=_f32)
    return acc


def _matmul1_body(aggA_ref, aggB_ref, nd0_ref, nd1_ref, w0_ref, w1_ref,
                  bs_ref, ns0_ref, ns1_ref, f0_ref, f1_ref):
    acc = _mm_block(aggA_ref, aggB_ref, nd0_ref, nd1_ref, w0_ref, w1_ref, bs_ref)
    f0_ref[...] = acc * ns0_ref[:, :1]
    f1_ref[...] = acc * ns1_ref[:, :1]


def _matmul2_body(aggA_ref, aggB_ref, nd0_ref, nd1_ref, w0_ref, w1_ref,
                  bs_ref, h_ref):
    h_ref[...] = _mm_block(aggA_ref, aggB_ref, nd0_ref, nd1_ref, w0_ref,
                           w1_ref, bs_ref)


_mm_in_specs = [
    pl.BlockSpec((BR, D), lambda i: (i, 0)),
    pl.BlockSpec((BR, D), lambda i: (i, 0)),
    pl.BlockSpec((BR, SW), lambda i: (i, 0)),
    pl.BlockSpec((BR, SW), lambda i: (i, 0)),
    pl.BlockSpec((D, D), lambda i: (0, 0)),
    pl.BlockSpec((D, D), lambda i: (0, 0)),
    pl.BlockSpec((1, D), lambda i: (0, 0)),
]

_matmul1 = pl.pallas_call(
    _matmul1_body,
    grid=(GRID,),
    in_specs=_mm_in_specs + [
        pl.BlockSpec((BR, SW), lambda i: (i, 0)),
        pl.BlockSpec((BR, SW), lambda i: (i, 0)),
    ],
    out_specs=[pl.BlockSpec((BR, D), lambda i: (i, 0))] * 2,
    out_shape=[jax.ShapeDtypeStruct((N, D), _f32)] * 2,
)

_matmul2 = pl.pallas_call(
    _matmul2_body,
    grid=(GRID,),
    in_specs=_mm_in_specs,
    out_specs=[pl.BlockSpec((BR, D), lambda i: (i, 0))],
    out_shape=[jax.ShapeDtypeStruct((N, D), _f32)],
)


def kernel(x, edge_index_r0, edge_index_r1, W_r0, b_r0, W_r1, b_r1):
    src0, dst0 = edge_index_r0[0], edge_index_r0[1]
    src1, dst1 = edge_index_r1[0], edge_index_r1[1]

    def padto(a, n, v):
        return jnp.concatenate(
            [a.astype(_i32), jnp.full((n - E,), v, _i32)])

    degidx = jnp.stack([padto(src0, EPD, N), padto(dst0, EPD, N),
                        padto(src1, EPD, N), padto(dst1, EPD, N)])
    degidx = degidx.reshape(4 * EPD)
    degout, _part = _degrees(degidx)

    deg = degout.reshape(4, NP)[:, :N]
    nrm = lax.rsqrt(jnp.clip(deg, 1.0, None))
    ns0e = jnp.broadcast_to(nrm[0][:, None], (N, SW))
    nd0e = jnp.broadcast_to(nrm[1][:, None], (N, SW))
    ns1e = jnp.broadcast_to(nrm[2][:, None], (N, SW))
    nd1e = jnp.broadcast_to(nrm[3][:, None], (N, SW))

    srcs = jnp.stack([padto(src0, EP, 0), padto(src1, EP, 0)]) * 4
    gidx = (srcs[:, None, :] +
            jnp.arange(NSLAB, dtype=_i32)[None, :, None])
    gidx = gidx.reshape(2, NSLAB, 16, NG, GT)
    dstidx = jnp.stack([padto(dst0, EP, N),
                        padto(dst1, EP, N)]).reshape(2, 16, NG, GT)
    zrows = jnp.zeros((GT, SW), _f32)

    w0r = W_r0
    w1r = W_r1
    bsum = (b_r0 + b_r1).reshape(1, D)

    f0, f1 = _featscale(x, ns0e, ns1e)
    aggA, aggB = _aggregate(f0.reshape(N * NSLAB, SW),
                            f1.reshape(N * NSLAB, SW),
                            gidx, dstidx, zrows)
    f0b, f1b = _matmul1(aggA, aggB, nd0e, nd1e, w0r, w1r, bsum, ns0e, ns1e)
    aggA2, aggB2 = _aggregate(f0b.reshape(N * NSLAB, SW),
                              f1b.reshape(N * NSLAB, SW),
                              gidx, dstidx, zrows)
    (h2,) = _matmul2(aggA2, aggB2, nd0e, nd1e, w0r, w1r, bsum)
    return h2


# speedup vs baseline: 1.6910x; 1.6910x over previous
import functools

import jax
import jax.numpy as jnp
from jax import lax
from jax.experimental import pallas as pl
from jax.experimental.pallas import tpu as pltpu
from jax.experimental.pallas import tpu_sc as plsc

N = 50000
D = 128
E = 310000
SW = 32
NSLAB = 4

GT = 128
NG = 152
IBG = 8
NCH = NG // IBG
NBUF = 4
PT = NG * GT
EP = 16 * PT
TPT = 3200
ACC = 16 * TPT

DPT = 19456
EPD = 16 * DPT
NP = 65536
NPT = NP // 16

BR = 512
GRID = (N + BR - 1) // BR

_mesh = plsc.VectorSubcoreMesh(core_axis_name="c", subcore_axis_name="s")
_sc_params = pltpu.CompilerParams(needs_layout_passes=False,
                                  use_tc_tiling_on_sc=False)
_f32 = jnp.float32
_i32 = jnp.int32


@functools.partial(
    pl.kernel,
    out_type=(jax.ShapeDtypeStruct((4 * NP,), _f32),
              jax.ShapeDtypeStruct((64 * NP,), _f32)),
    mesh=_mesh,
    compiler_params=_sc_params,
    scratch_types=[
        pltpu.VMEM((NP,), _f32),
        pltpu.VMEM((DPT,), _i32),
        pltpu.VMEM((NPT,), _f32),
        pltpu.VMEM((NPT,), _f32),
    ],
)
def _degrees(degidx_hbm, degout_hbm, part_hbm, cnt, idxb, acc, tmp):
    c = lax.axis_index("c")
    s = lax.axis_index("s")
    zeros16 = jnp.zeros((16,), _f32)
    ones16 = jnp.ones((16,), _f32)
    for q in range(2):
        p = 2 * c + q

        def zero_body(i, _):
            cnt[pl.ds(i * 16, 16)] = zeros16
            return 0

        lax.fori_loop(0, NP // 16, zero_body, 0)
        pltpu.sync_copy(degidx_hbm.at[pl.ds((p * 16 + s) * DPT, DPT)], idxb)

        def scat_body(j, _):
            idx16 = idxb[pl.ds(j * 16, 16)]
            plsc.addupdate_scatter(cnt, [idx16], ones16)
            return 0

        lax.fori_loop(0, DPT // 16, scat_body, 0)
        pltpu.sync_copy(cnt, part_hbm.at[pl.ds((p * 16 + s) * NP, NP)])
        plsc.subcore_barrier()
        pltpu.sync_copy(part_hbm.at[pl.ds((p * 16) * NP + s * NPT, NPT)], acc)
        for u in range(1, 16):
            pltpu.sync_copy(
                part_hbm.at[pl.ds((p * 16 + u) * NP + s * NPT, NPT)], tmp)

            def red_body(i, _):
                sl = pl.ds(i * 16, 16)
                acc[sl] = acc[sl] + tmp[sl]
                return 0

            lax.fori_loop(0, NPT // 16, red_body, 0)
        pltpu.sync_copy(acc, degout_hbm.at[pl.ds(p * NP + s * NPT, NPT)])


@functools.partial(
    pl.kernel,
    out_type=jax.ShapeDtypeStruct((ACC, D), _f32),
    mesh=_mesh,
    compiler_params=_sc_params,
    scratch_types=[
        pltpu.VMEM((IBG, GT), _i32),
        pltpu.VMEM((IBG, GT), _i32),
        pltpu.VMEM((NBUF, GT, SW), _f32),
        pltpu.VMEM((GT, SW), _f32),
        pltpu.VMEM_SHARED((ACC, SW), _f32),
    ] + [pltpu.SemaphoreType.DMA] * NBUF,
)
def _aggregate(feat, gidx, dstidx, zrows, out,
               gbuf, dbuf, rows, zb, sacc, *sems):
    c = lax.axis_index("c")
    s = lax.axis_index("s")
    pltpu.sync_copy(zrows, zb)
    for q in range(2):
        slab = 2 * c + q

        def zero_body(j, _):
            pltpu.sync_copy(zb, sacc.at[pl.ds(s * TPT + j * GT, GT)])
            return 0

        lax.fori_loop(0, TPT // GT, zero_body, 0)
        plsc.subcore_barrier()

        def chunk_body(ch, _):
            pltpu.sync_copy(gidx.at[slab, s, pl.ds(ch * IBG, IBG)], gbuf)
            pltpu.sync_copy(dstidx.at[s, pl.ds(ch * IBG, IBG)], dbuf)
            descs = [None] * IBG
            for g in range(NBUF):
                descs[g] = pltpu.async_copy(
                    feat.at[gbuf.at[g]], rows.at[g], sems[g])
            for g in range(IBG):
                b = g % NBUF
                descs[g].wait()
                pltpu.sync_copy(rows.at[b], sacc.at[dbuf.at[g]], add=True)
                if g + NBUF < IBG:
                    descs[g + NBUF] = pltpu.async_copy(
                        feat.at[gbuf.at[g + NBUF]], rows.at[b], sems[b])
            return 0

        lax.fori_loop(0, NCH, chunk_body, 0)
        plsc.subcore_barrier()

        def out_body(j, _):
            pltpu.sync_copy(sacc.at[pl.ds(s * TPT + j * GT, GT)],
                            out.at[pl.ds(s * TPT + j * GT, GT),
                                   pl.ds(slab * SW, SW)])
            return 0

        lax.fori_loop(0, TPT // GT, out_body, 0)
        plsc.subcore_barrier()


def _featscale_body(x_ref, ns0_ref, ns1_ref, f0_ref, f1_ref):
    xb = x_ref[...]
    f0_ref[...] = xb * ns0_ref[:, :1]
    f1_ref[...] = xb * ns1_ref[:, :1]


_featscale = pl.pallas_call(
    _featscale_body,
    grid=(GRID,),
    in_specs=[
        pl.BlockSpec((BR, D), lambda i: (i, 0)),
        pl.BlockSpec((BR, SW), lambda i: (i, 0)),
        pl.BlockSpec((BR, SW), lambda i: (i, 0)),
    ],
    out_specs=[
        pl.BlockSpec((BR, D), lambda i: (i, 0)),
        pl.BlockSpec((BR, D), lambda i: (i, 0)),
    ],
    out_shape=[
        jax.ShapeDtypeStruct((N, D), _f32),
        jax.ShapeDtypeStruct((N, D), _f32),
    ],
)


def _dot(a, w):
    return jnp.dot(a, w, precision=lax.Precision.HIGHEST,
                   preferred_element_type=_f32)


def _mm_part_body(aggA_ref, nd0_ref, w0_ref, bs_ref, t_ref):
    t_ref[...] = (jnp.broadcast_to(bs_ref[...], (BR, D)) +
                  _dot(aggA_ref[...] * nd0_ref[:, :1], w0_ref[...]))


def _mm_fin1_body(t_ref, aggB_ref, nd1_ref, w1_ref, ns0_ref, ns1_ref,
                  f0_ref, f1_ref):
    acc = t_ref[...] + _dot(aggB_ref[...] * nd1_ref[:, :1], w1_ref[...])
    f0_ref[...] = acc * ns0_ref[:, :1]
    f1_ref[...] = acc * ns1_ref[:, :1]


def _mm_fin2_body(t_ref, aggB_ref, nd1_ref, w1_ref, h_ref):
    h_ref[...] = t_ref[...] + _dot(aggB_ref[...] * nd1_ref[:, :1],
                                   w1_ref[...])


_bd = pl.BlockSpec((BR, D), lambda i: (i, 0))
_bs = pl.BlockSpec((BR, SW), lambda i: (i, 0))
_bw = pl.BlockSpec((D, D), lambda i: (0, 0))

_mm_part = pl.pallas_call(
    _mm_part_body,
    grid=(GRID,),
    in_specs=[_bd, _bs, _bw, pl.BlockSpec((1, D), lambda i: (0, 0))],
    out_specs=[_bd],
    out_shape=[jax.ShapeDtypeStruct((N, D), _f32)],
)

_mm_fin1 = pl.pallas_call(
    _mm_fin1_body,
    grid=(GRID,),
    in_specs=[_bd, _bd, _bs, _bw, _bs, _bs],
    out_specs=[_bd, _bd],
    out_shape=[jax.ShapeDtypeStruct((N, D), _f32)] * 2,
)

_mm_fin2 = pl.pallas_call(
    _mm_fin2_body,
    grid=(GRID,),
    in_specs=[_bd, _bd, _bs, _bw],
    out_specs=[_bd],
    out_shape=[jax.ShapeDtypeStruct((N, D), _f32)],
)


def kernel(x, edge_index_r0, edge_index_r1, W_r0, b_r0, W_r1, b_r1):
    src0, dst0 = edge_index_r0[0], edge_index_r0[1]
    src1, dst1 = edge_index_r1[0], edge_index_r1[1]

    def padto(a, n, v):
        return jnp.concatenate(
            [a.astype(_i32), jnp.full((n - E,), v, _i32)])

    degidx = jnp.stack([padto(src0, EPD, N), padto(dst0, EPD, N),
                        padto(src1, EPD, N), padto(dst1, EPD, N)])
    degidx = degidx.reshape(4 * EPD)
    degout, _part = _degrees(degidx)

    deg = degout.reshape(4, NP)[:, :N]
    nrm = lax.rsqrt(jnp.clip(deg, 1.0, None))
    ns0e = jnp.broadcast_to(nrm[0][:, None], (N, SW))
    nd0e = jnp.broadcast_to(nrm[1][:, None], (N, SW))
    ns1e = jnp.broadcast_to(nrm[2][:, None], (N, SW))
    nd1e = jnp.broadcast_to(nrm[3][:, None], (N, SW))

    slabs = jnp.arange(NSLAB, dtype=_i32)[:, None]
    g0 = ((padto(src0, EP, 0) * 4)[None, :] + slabs).reshape(NSLAB, 16, NG, GT)
    g1 = ((padto(src1, EP, 0) * 4)[None, :] + slabs).reshape(NSLAB, 16, NG, GT)
    d0 = padto(dst0, EP, N).reshape(16, NG, GT)
    d1 = padto(dst1, EP, N).reshape(16, NG, GT)
    zrows = jnp.zeros((GT, SW), _f32)
    bsum = (b_r0 + b_r1).reshape(1, D)

    f0, f1 = _featscale(x, ns0e, ns1e)
    aggA = _aggregate(f0.reshape(N * NSLAB, SW), g0, d0, zrows)
    (t1,) = _mm_part(aggA, nd0e, W_r0, bsum)
    aggB = _aggregate(f1.reshape(N * NSLAB, SW), g1, d1, zrows)
    f0b, f1b = _mm_fin1(t1, aggB, nd1e, W_r1, ns0e, ns1e)
    aggA2 = _aggregate(f0b.reshape(N * NSLAB, SW), g0, d0, zrows)
    (t2,) = _mm_part(aggA2, nd0e, W_r0, bsum)
    aggB2 = _aggregate(f1b.reshape(N * NSLAB, SW), g1, d1, zrows)
    (h2,) = _mm_fin2(t2, aggB2, nd1e, W_r1)
    return h2
